# Initial kernel scaffold; baseline (speedup 1.0000x reference)
import jax
import jax.numpy as jnp
from jax.experimental import pallas as pl


def kernel(x, edge_index, batch, emb, W1, b1, W2, b2):
    raise NotImplementedError("write your pallas kernel here")


# speedup vs baseline: 33.4034x; 33.4034x over previous
import functools

import jax
import jax.numpy as jnp
from jax import lax
from jax.experimental import pallas as pl
from jax.experimental.pallas import tpu as pltpu
from jax.experimental.pallas import tpu_sc as plsc

N_NODES = 100000
N_EDGES = 1600000
NUM_GRAPHS = 512
F0 = 16
F1 = 32
F2 = 41

NC = 2
NS = 16
NW = NC * NS
L = 16

NPAD = 102400
NODES_PER_TILE = NPAD // NW
NODES_PER_CTILE = NPAD // NS
EDGES_PER_CORE = N_EDGES // NC
E_TILE_K1 = EDGES_PER_CORE // NS
E_TILE_K4 = N_EDGES // NS
ECHUNK = 2000
EPCHUNK = 1000
NCHUNK = 640

_mesh = plsc.VectorSubcoreMesh(core_axis_name="c", subcore_axis_name="s")


def _zero_vmem(ref, n_rows):
    zeros = jnp.zeros((L,), jnp.float32)

    def body(i, _):
        ref[i, :] = zeros
        return 0

    lax.fori_loop(0, n_rows, body, 0)


def _rsqrt16(d):
    i = lax.bitcast_convert_type(d, jnp.int32)
    i = 0x5F3759DF - lax.shift_right_logical(i, 1)
    y = lax.bitcast_convert_type(i, jnp.float32)
    for _ in range(3):
        y = y * (1.5 - 0.5 * d * y * y)
    return y


@functools.partial(
    pl.kernel,
    compiler_params=pltpu.CompilerParams(use_tc_tiling_on_sc=False),
    out_type=jax.ShapeDtypeStruct((NC, NPAD), jnp.float32),
    mesh=_mesh,
    scratch_types=dict(
        deg_sh=pltpu.VMEM_SHARED((NPAD,), jnp.float32),
        idx_v=pltpu.VMEM((ECHUNK,), jnp.int32),
        ones_v=pltpu.VMEM((ECHUNK,), jnp.float32),
        zbuf=pltpu.VMEM((NODES_PER_CTILE,), jnp.float32),
    ),
)
def _deg_kernel(dst_hbm, deg_out, deg_sh, idx_v, ones_v, zbuf):
    cid = lax.axis_index("c")
    sid = lax.axis_index("s")

    zeros = jnp.zeros((L,), jnp.float32)
    ones = jnp.ones((L,), jnp.float32)

    def fill(i, _):
        zbuf[pl.ds(i * L, L)] = zeros
        return 0

    lax.fori_loop(0, NODES_PER_CTILE // L, fill, 0)

    def fill_ones(i, _):
        ones_v[pl.ds(i * L, L)] = ones
        return 0

    lax.fori_loop(0, ECHUNK // L, fill_ones, 0)

    pltpu.sync_copy(zbuf, deg_sh.at[pl.ds(sid * NODES_PER_CTILE, NODES_PER_CTILE)])
    plsc.subcore_barrier()

    deg_flat = deg_sh
    ebase = cid * EDGES_PER_CORE + sid * E_TILE_K1
    for k in range(E_TILE_K1 // ECHUNK):
        pltpu.sync_copy(dst_hbm.at[pl.ds(ebase + k * ECHUNK, ECHUNK)], idx_v)
        pltpu.sync_copy(ones_v, deg_flat.at[idx_v], add=True)

    plsc.subcore_barrier()
    pltpu.sync_copy(
        deg_flat.at[pl.ds(sid * NODES_PER_CTILE, NODES_PER_CTILE)],
        deg_out.at[cid, pl.ds(sid * NODES_PER_CTILE, NODES_PER_CTILE)],
    )


@functools.partial(
    pl.kernel,
    compiler_params=pltpu.CompilerParams(use_tc_tiling_on_sc=False),
    out_type=(
        jax.ShapeDtypeStruct((NPAD,), jnp.float32),
        jax.ShapeDtypeStruct((NPAD, F0), jnp.float32),
    ),
    mesh=_mesh,
    scratch_types=dict(
        d0_v=pltpu.VMEM((NCHUNK,), jnp.float32),
        d1_v=pltpu.VMEM((NCHUNK,), jnp.float32),
        x_v=pltpu.VMEM((NCHUNK,), jnp.int32),
        dinv_v=pltpu.VMEM((NCHUNK,), jnp.float32),
        rows_v=pltpu.VMEM((NCHUNK, F0), jnp.float32),
        sem=pltpu.SemaphoreType.DMA,
    ),
)
def _dinv_h0_kernel(deg_hbm, x_hbm, emb_hbm, dinv_out, h0_out,
                    d0_v, d1_v, x_v, dinv_v, rows_v, sem):
    cid = lax.axis_index("c")
    sid = lax.axis_index("s")
    wid = sid * NC + cid
    base = wid * NODES_PER_TILE

    for k in range(NODES_PER_TILE // NCHUNK):
        nb = base + k * NCHUNK
        pltpu.sync_copy(deg_hbm.at[0, pl.ds(nb, NCHUNK)], d0_v)
        pltpu.sync_copy(deg_hbm.at[1, pl.ds(nb, NCHUNK)], d1_v)
        pltpu.sync_copy(x_hbm.at[pl.ds(nb, NCHUNK)], x_v)
        pltpu.async_copy(emb_hbm.at[x_v], rows_v, sem).wait()

        def vbody(i, _):
            sl = pl.ds(i * L, L)
            d = d0_v[sl] + d1_v[sl] + 1.0
            dinv_v[sl] = _rsqrt16(d)
            return 0

        lax.fori_loop(0, NCHUNK // L, vbody, 0)

        pltpu.sync_copy(dinv_v, dinv_out.at[pl.ds(nb, NCHUNK)])
        pltpu.sync_copy(rows_v, h0_out.at[pl.ds(nb, NCHUNK)])


@functools.partial(
    pl.kernel,
    compiler_params=pltpu.CompilerParams(use_tc_tiling_on_sc=False),
    out_type=jax.ShapeDtypeStruct((NC, NPAD, F0), jnp.float32),
    mesh=_mesh,
    scratch_types=dict(
        acc_sh=pltpu.VMEM_SHARED((NPAD, F0), jnp.float32),
        sidx_v=pltpu.VMEM((EPCHUNK,), jnp.int32),
        didx_v=pltpu.VMEM((EPCHUNK,), jnp.int32),
        rows_v=pltpu.VMEM((EPCHUNK, F0), jnp.float32),
        zbuf=pltpu.VMEM((100, F0), jnp.float32),
        sem=pltpu.SemaphoreType.DMA,
    ),
)
def _prop1_kernel(src_hbm, dst_hbm, g0_hbm, acc_out,
                  acc_sh, sidx_v, didx_v, rows_v, zbuf, sem):
    cid = lax.axis_index("c")
    sid = lax.axis_index("s")

    _zero_vmem(zbuf, 100)
    for z in range(NODES_PER_CTILE // 100):
        pltpu.sync_copy(zbuf, acc_sh.at[pl.ds(sid * NODES_PER_CTILE + z * 100, 100)])
    plsc.subcore_barrier()

    ebase = cid * EDGES_PER_CORE + sid * E_TILE_K1
    for k in range(E_TILE_K1 // EPCHUNK):
        pltpu.sync_copy(src_hbm.at[pl.ds(ebase + k * EPCHUNK, EPCHUNK)], sidx_v)
        pltpu.sync_copy(dst_hbm.at[pl.ds(ebase + k * EPCHUNK, EPCHUNK)], didx_v)
        pltpu.async_copy(g0_hbm.at[sidx_v], rows_v, sem).wait()
        pltpu.sync_copy(rows_v, acc_sh.at[didx_v], add=True)

    plsc.subcore_barrier()
    pltpu.sync_copy(
        acc_sh.at[pl.ds(sid * NODES_PER_CTILE, NODES_PER_CTILE)],
        acc_out.at[cid, pl.ds(sid * NODES_PER_CTILE, NODES_PER_CTILE)],
    )


@functools.partial(
    pl.kernel,
    compiler_params=pltpu.CompilerParams(use_tc_tiling_on_sc=False),
    out_type=jax.ShapeDtypeStruct((NC, NPAD, F0), jnp.float32),
    mesh=_mesh,
    scratch_types=dict(
        acc_sh=pltpu.VMEM_SHARED((NPAD, F0), jnp.float32),
        sidx_v=pltpu.VMEM((EPCHUNK,), jnp.int32),
        didx_v=pltpu.VMEM((EPCHUNK,), jnp.int32),
        rows_v=pltpu.VMEM((EPCHUNK, F0), jnp.float32),
        zbuf=pltpu.VMEM((100, F0), jnp.float32),
        sem=pltpu.SemaphoreType.DMA,
    ),
)
def _prop2_kernel(src_hbm, dst_hbm, g2a_hbm, g2b_hbm, acc_out,
                  acc_sh, sidx_v, didx_v, rows_v, zbuf, sem):
    cid = lax.axis_index("c")
    sid = lax.axis_index("s")

    _zero_vmem(zbuf, 100)
    for z in range(NODES_PER_CTILE // 100):
        pltpu.sync_copy(zbuf, acc_sh.at[pl.ds(sid * NODES_PER_CTILE + z * 100, 100)])
    plsc.subcore_barrier()

    ebase = sid * E_TILE_K4
    for k in range(E_TILE_K4 // EPCHUNK):
        pltpu.sync_copy(src_hbm.at[pl.ds(ebase + k * EPCHUNK, EPCHUNK)], sidx_v)
        pltpu.sync_copy(dst_hbm.at[pl.ds(ebase + k * EPCHUNK, EPCHUNK)], didx_v)

        @pl.when(cid == 0)
        def _():
            pltpu.async_copy(g2a_hbm.at[sidx_v], rows_v, sem).wait()

        @pl.when(cid == 1)
        def _():
            pltpu.async_copy(g2b_hbm.at[sidx_v], rows_v, sem).wait()

        pltpu.sync_copy(rows_v, acc_sh.at[didx_v], add=True)

    plsc.subcore_barrier()
    pltpu.sync_copy(
        acc_sh.at[pl.ds(sid * NODES_PER_CTILE, NODES_PER_CTILE)],
        acc_out.at[cid, pl.ds(sid * NODES_PER_CTILE, NODES_PER_CTILE)],
    )


@functools.partial(
    pl.kernel,
    compiler_params=pltpu.CompilerParams(use_tc_tiling_on_sc=False),
    out_type=(
        jax.ShapeDtypeStruct((NC, NUM_GRAPHS, F1), jnp.float32),
        jax.ShapeDtypeStruct((NC, NUM_GRAPHS, L), jnp.float32),
    ),
    mesh=_mesh,
    scratch_types=dict(
        seg_sh=pltpu.VMEM_SHARED((NUM_GRAPHS, F1), jnp.float32),
        cnt_sh=pltpu.VMEM_SHARED((NUM_GRAPHS, L), jnp.float32),
        seg_v=pltpu.VMEM((NUM_GRAPHS, F1), jnp.float32),
        cnt_v=pltpu.VMEM((NUM_GRAPHS, L), jnp.float32),
        p_v=pltpu.VMEM((NCHUNK, F1), jnp.float32),
        b_v=pltpu.VMEM((NCHUNK,), jnp.int32),
        gid_v=pltpu.VMEM((NUM_GRAPHS,), jnp.int32),
        zbuf=pltpu.VMEM((NUM_GRAPHS, F1), jnp.float32),
        zcnt=pltpu.VMEM((NUM_GRAPHS, L), jnp.float32),
    ),
)
def _readout_kernel(prop2_hbm, batch_hbm, seg_out, cnt_out,
                    seg_sh, cnt_sh, seg_v, cnt_v, p_v, b_v, gid_v, zbuf, zcnt):
    cid = lax.axis_index("c")
    sid = lax.axis_index("s")
    wid = sid * NC + cid
    base = wid * NODES_PER_TILE

    zrow = jnp.zeros((L,), jnp.float32)

    def zseg(i, _):
        seg_v[i, pl.ds(0, L)] = zrow
        seg_v[i, pl.ds(L, L)] = zrow
        cnt_v[i, :] = zrow
        zbuf[i, pl.ds(0, L)] = zrow
        zbuf[i, pl.ds(L, L)] = zrow
        zcnt[i, :] = zrow
        return 0

    lax.fori_loop(0, NUM_GRAPHS, zseg, 0)

    def ziota(i, _):
        gid_v[pl.ds(i * L, L)] = lax.iota(jnp.int32, L) + i * L
        return 0

    lax.fori_loop(0, NUM_GRAPHS // L, ziota, 0)

    @pl.when(sid == 0)
    def _():
        pltpu.sync_copy(zbuf, seg_sh)
        pltpu.sync_copy(zcnt, cnt_sh)

    ones = jnp.ones((L,), jnp.float32)
    for k in range(NODES_PER_TILE // NCHUNK):
        nb = base + k * NCHUNK
        pltpu.sync_copy(prop2_hbm.at[pl.ds(nb, NCHUNK)], p_v)
        pltpu.sync_copy(batch_hbm.at[pl.ds(nb, NCHUNK)], b_v)
        n_groups = jnp.clip(N_NODES - nb, 0, NCHUNK) // L

        def gbody(j, _):
            b16 = b_v[pl.ds(j * L, L)]
            for l in range(L):
                g = b16[l]
                n = j * L + l
                plsc.addupdate(seg_v.at[g, pl.ds(0, L)], p_v[n, pl.ds(0, L)])
                plsc.addupdate(seg_v.at[g, pl.ds(L, L)], p_v[n, pl.ds(L, L)])
                plsc.addupdate(cnt_v.at[g], ones)
            return 0

        lax.fori_loop(0, n_groups, gbody, 0)

    plsc.subcore_barrier()
    pltpu.sync_copy(seg_v, seg_sh.at[gid_v], add=True)
    pltpu.sync_copy(cnt_v, cnt_sh.at[gid_v], add=True)
    plsc.subcore_barrier()

    @pl.when(sid == 0)
    def _():
        pltpu.sync_copy(seg_sh, seg_out.at[cid])
        pltpu.sync_copy(cnt_sh, cnt_out.at[cid])


TC_BLK = 2048


def _tc_scale_body(h0_ref, dinv_ref, g0_ref):
    g0_ref[...] = h0_ref[...] * dinv_ref[...]


def _tc_a_body(acc_ref, g0_ref, dinv_ref, w1_ref, b1_ref, g2a_ref, g2b_ref):
    a = acc_ref[0] + acc_ref[1] + g0_ref[...]
    pre = a * dinv_ref[...]
    h = jnp.dot(pre, w1_ref[...], preferred_element_type=jnp.float32)
    h = jnp.maximum(h + b1_ref[...], 0.0)
    g2 = h * dinv_ref[...]
    g2a_ref[...] = g2[:, :F0]
    g2b_ref[...] = g2[:, F0:]


def _tc_b_body(acc_ref, g2a_ref, g2b_ref, dinv_ref, out_ref):
    pa = (acc_ref[0] + g2a_ref[...]) * dinv_ref[...]
    pb = (acc_ref[1] + g2b_ref[...]) * dinv_ref[...]
    out_ref[...] = jnp.concatenate([pa, pb], axis=1)


def _tc_c_body(seg_ref, cnt_ref, w2_ref, b2_ref, out_ref):
    s = seg_ref[0] + seg_ref[1]
    c = cnt_ref[0, :, 0:1] + cnt_ref[1, :, 0:1]
    out_ref[...] = (
        jnp.dot(s, w2_ref[...], preferred_element_type=jnp.float32)
        + c * b2_ref[...]
    )


def kernel(x, edge_index, batch, emb, W1, b1, W2, b2):
    x = jnp.squeeze(x, axis=-1).astype(jnp.int32)
    src = edge_index[0].astype(jnp.int32)
    dst = edge_index[1].astype(jnp.int32)
    batch = batch.astype(jnp.int32)

    x_pad = jnp.pad(x, (0, NPAD - N_NODES))
    batch_pad = jnp.pad(batch, (0, NPAD - N_NODES))

    deg_parts = _deg_kernel(dst)
    dinv, h0 = _dinv_h0_kernel(deg_parts, x_pad, emb)

    dinv2d = dinv.reshape(NPAD, 1)
    grid = NPAD // TC_BLK
    g0 = pl.pallas_call(
        _tc_scale_body,
        grid=(grid,),
        in_specs=[
            pl.BlockSpec((TC_BLK, F0), lambda i: (i, 0)),
            pl.BlockSpec((TC_BLK, 1), lambda i: (i, 0)),
        ],
        out_specs=pl.BlockSpec((TC_BLK, F0), lambda i: (i, 0)),
        out_shape=jax.ShapeDtypeStruct((NPAD, F0), jnp.float32),
    )(h0, dinv2d)

    acc1 = _prop1_kernel(src, dst, g0)
    g2a, g2b = pl.pallas_call(
        _tc_a_body,
        grid=(grid,),
        in_specs=[
            pl.BlockSpec((NC, TC_BLK, F0), lambda i: (0, i, 0)),
            pl.BlockSpec((TC_BLK, F0), lambda i: (i, 0)),
            pl.BlockSpec((TC_BLK, 1), lambda i: (i, 0)),
            pl.BlockSpec((F0, F1), lambda i: (0, 0)),
            pl.BlockSpec((1, F1), lambda i: (0, 0)),
        ],
        out_specs=[
            pl.BlockSpec((TC_BLK, F0), lambda i: (i, 0)),
            pl.BlockSpec((TC_BLK, F0), lambda i: (i, 0)),
        ],
        out_shape=[
            jax.ShapeDtypeStruct((NPAD, F0), jnp.float32),
            jax.ShapeDtypeStruct((NPAD, F0), jnp.float32),
        ],
    )(acc1, g0, dinv2d, W1, b1.reshape(1, F1))

    acc2 = _prop2_kernel(src, dst, g2a, g2b)

    prop2 = pl.pallas_call(
        _tc_b_body,
        grid=(grid,),
        in_specs=[
            pl.BlockSpec((NC, TC_BLK, F0), lambda i: (0, i, 0)),
            pl.BlockSpec((TC_BLK, F0), lambda i: (i, 0)),
            pl.BlockSpec((TC_BLK, F0), lambda i: (i, 0)),
            pl.BlockSpec((TC_BLK, 1), lambda i: (i, 0)),
        ],
        out_specs=pl.BlockSpec((TC_BLK, F1), lambda i: (i, 0)),
        out_shape=jax.ShapeDtypeStruct((NPAD, F1), jnp.float32),
    )(acc2, g2a, g2b, dinv2d)

    seg_parts, cnt_parts = _readout_kernel(prop2, batch_pad)

    out = pl.pallas_call(
        _tc_c_body,
        out_shape=jax.ShapeDtypeStruct((NUM_GRAPHS, F2), jnp.float32),
    )(seg_parts, cnt_parts, W2, b2.reshape(1, F2))
    return out
